# baseline (speedup 1.0000x reference)
import jax
import jax.numpy as jnp
from jax.experimental import pallas as pl
from jax.experimental.pallas import tpu as pltpu


def _fused_kernel(x_ref, wc_ref, bc_ref, w2_ref, b2_ref, w3_ref, b3_ref,
                  wv2_ref, bv2_ref, logits_ref, values_ref, *, d1):
    x = x_ref[...]
    hc = jnp.maximum(
        jax.lax.dot_general(x, wc_ref[...], (((1,), (0,)), ((), ())),
                            preferred_element_type=jnp.float32) + bc_ref[...],
        0.0)
    h1 = hc[:, :d1]
    v1 = hc[:, d1:]
    h2 = jnp.maximum(
        jax.lax.dot_general(h1, w2_ref[...], (((1,), (0,)), ((), ())),
                            preferred_element_type=jnp.float32) + b2_ref[...],
        0.0)
    logits_ref[...] = jax.lax.dot_general(
        h2, w3_ref[...], (((1,), (0,)), ((), ())),
        preferred_element_type=jnp.float32) + b3_ref[...]
    values_ref[...] = jax.lax.dot_general(
        v1, wv2_ref[...], (((1,), (0,)), ((), ())),
        preferred_element_type=jnp.float32) + bv2_ref[...]


def kernel(hidden_states, W1, b1, W2, b2, W3, b3, Wv1, bv1, Wv2, bv2):
    B, S, H = hidden_states.shape
    N = B * S
    d1 = W1.shape[1]
    d2 = W2.shape[1]
    ne = W3.shape[1]

    flat = hidden_states.reshape(N, H)
    Wc = jnp.concatenate([W1, Wv1], axis=1)
    bc = jnp.concatenate([b1, bv1]).reshape(1, -1)

    TILE = 2048
    grid = (N // TILE,)

    import functools
    body = functools.partial(_fused_kernel, d1=d1)

    logits, values = pl.pallas_call(
        body,
        grid=grid,
        in_specs=[
            pl.BlockSpec((TILE, H), lambda i: (i, 0)),
            pl.BlockSpec((H, 2 * d1), lambda i: (0, 0)),
            pl.BlockSpec((1, 2 * d1), lambda i: (0, 0)),
            pl.BlockSpec((d1, d2), lambda i: (0, 0)),
            pl.BlockSpec((1, d2), lambda i: (0, 0)),
            pl.BlockSpec((d2, ne), lambda i: (0, 0)),
            pl.BlockSpec((1, ne), lambda i: (0, 0)),
            pl.BlockSpec((d1, 1), lambda i: (0, 0)),
            pl.BlockSpec((1, 1), lambda i: (0, 0)),
        ],
        out_specs=[
            pl.BlockSpec((TILE, ne), lambda i: (i, 0)),
            pl.BlockSpec((TILE, 1), lambda i: (i, 0)),
        ],
        out_shape=[
            jax.ShapeDtypeStruct((N, ne), jnp.float32),
            jax.ShapeDtypeStruct((N, 1), jnp.float32),
        ],
        compiler_params=pltpu.CompilerParams(
            dimension_semantics=("arbitrary",),
        ),
    )(flat, Wc, bc, W2, b2.reshape(1, -1), W3, b3.reshape(1, -1),
      Wv2, bv2.reshape(1, -1))

    return (logits.reshape(B, S, ne), values.reshape(B, S, 1))
